# baseline (speedup 1.0000x reference)
import functools

import jax
import jax.numpy as jnp
from jax import lax
from jax.experimental import pallas as pl
from jax.experimental.pallas import tpu as pltpu
from jax.experimental.pallas import tpu_sc as plsc

_B = 4096
_N = 8192
_NC = 2
_NS = 16
_NW = _NC * _NS
_GROUPS = _B // (_NW * 16)
_SEG = 2048
_NSEG = _N // _SEG
_UNROLL = 8


def _sc_body(x_hbm, y_hbm, in_v, out_v, sem):
    wid = lax.axis_index("s") * _NC + lax.axis_index("c")
    lane = lax.broadcasted_iota(jnp.int32, (16,), 0)
    rowbase = lane * _SEG

    def per_group(g, _):
        r0 = (wid * _GROUPS + g) * 16

        def per_seg(si, carry):
            c0 = (_NSEG - 1 - si) * _SEG
            in_cps = [
                pltpu.async_copy(
                    x_hbm.at[r0 + rl, pl.ds(c0, _SEG)],
                    in_v.at[pl.ds(rl * _SEG, _SEG)],
                    sem,
                )
                for rl in range(16)
            ]
            for cp in in_cps:
                cp.wait()

            def head(hi, carry):
                j = (_SEG + 14 - hi) - lane
                m = j <= _SEG - 1
                idx = rowbase + jnp.minimum(j, _SEG - 1)
                v = plsc.load_gather(in_v, [idx], mask=m)
                carry = carry + jnp.where(m, v, jnp.float32(0))
                plsc.store_scatter(out_v, [idx], carry, mask=m)
                return carry

            carry = lax.fori_loop(0, 15, head, carry)

            def main(_, state):
                carry, idx = state
                for k in range(_UNROLL):
                    cur = idx - k
                    v = plsc.load_gather(in_v, [cur])
                    carry = carry + v
                    plsc.store_scatter(out_v, [cur], carry)
                return carry, idx - _UNROLL

            carry, _ = lax.fori_loop(
                0, (_SEG - 16) // _UNROLL, main,
                (carry, rowbase + (_SEG - 1) - lane),
            )

            def tail(ti, carry):
                j = (15 - ti) - lane
                m = j >= 0
                idx = rowbase + jnp.maximum(j, 0)
                v = plsc.load_gather(in_v, [idx], mask=m)
                carry = carry + jnp.where(m, v, jnp.float32(0))
                plsc.store_scatter(out_v, [idx], carry, mask=m)
                return carry

            carry = lax.fori_loop(0, 16, tail, carry)

            out_cps = [
                pltpu.async_copy(
                    out_v.at[pl.ds(rl * _SEG, _SEG)],
                    y_hbm.at[r0 + rl, pl.ds(c0, _SEG)],
                    sem,
                )
                for rl in range(16)
            ]
            for cp in out_cps:
                cp.wait()
            return carry

        lax.fori_loop(0, _NSEG, per_seg, jnp.zeros((16,), jnp.float32))
        return 0

    lax.fori_loop(0, _GROUPS, per_group, 0)


def kernel(x):
    mesh = plsc.VectorSubcoreMesh(core_axis_name="c", subcore_axis_name="s")
    k = functools.partial(
        pl.kernel,
        mesh=mesh,
        out_type=jax.ShapeDtypeStruct((_B, _N), jnp.float32),
        scratch_types=[
            pltpu.VMEM((16 * _SEG,), jnp.float32),
            pltpu.VMEM((16 * _SEG,), jnp.float32),
            pltpu.SemaphoreType.DMA,
        ],
        compiler_params=pltpu.CompilerParams(needs_layout_passes=False),
    )(_sc_body)
    return k(x)


# speedup vs baseline: 1.9140x; 1.9140x over previous
import functools

import jax
import jax.numpy as jnp
from jax import lax
from jax.experimental import pallas as pl
from jax.experimental.pallas import tpu as pltpu
from jax.experimental.pallas import tpu_sc as plsc

_B = 4096
_N = 8192
_NC = 2
_NS = 16
_NW = _NC * _NS
_GROUPS = _B // (_NW * 16)
_SEG = 2048
_NSEG = _N // _SEG
_UNROLL = 8


def _sc_body(x_hbm, y_hbm, in_v, out_v, sem):
    wid = lax.axis_index("s") * _NC + lax.axis_index("c")
    lane = lax.broadcasted_iota(jnp.int32, (16,), 0)
    rowbase = lane * _SEG

    def per_group(g, _):
        r0 = (wid * _GROUPS + g) * 16

        def per_seg(si, carry):
            c0 = (_NSEG - 1 - si) * _SEG
            in_cps = [
                pltpu.async_copy(
                    x_hbm.at[r0 + rl, pl.ds(c0, _SEG)],
                    in_v.at[pl.ds(rl * _SEG, _SEG)],
                    sem,
                )
                for rl in range(16)
            ]
            for cp in in_cps:
                cp.wait()

            def head(hi, carry):
                j = (_SEG + 14 - hi) - lane
                m = j <= _SEG - 1
                idx = rowbase + jnp.minimum(j, _SEG - 1)
                v = plsc.load_gather(in_v, [idx], mask=m)
                carry = carry + jnp.where(m, v, jnp.float32(0))
                plsc.store_scatter(out_v, [idx], carry, mask=m)
                return carry

            carry = lax.fori_loop(0, 15, head, carry)

            def main(_, state):
                carry, idx = state
                vs = [plsc.load_gather(in_v, [idx - k]) for k in range(_UNROLL)]
                pref = [vs[0]]
                for k in range(1, _UNROLL):
                    pref.append(pref[-1] + vs[k])
                for k in range(_UNROLL):
                    plsc.store_scatter(out_v, [idx - k], carry + pref[k])
                return carry + pref[-1], idx - _UNROLL

            carry, _ = lax.fori_loop(
                0, (_SEG - 16) // _UNROLL, main,
                (carry, rowbase + (_SEG - 1) - lane),
            )

            def tail(ti, carry):
                j = (15 - ti) - lane
                m = j >= 0
                idx = rowbase + jnp.maximum(j, 0)
                v = plsc.load_gather(in_v, [idx], mask=m)
                carry = carry + jnp.where(m, v, jnp.float32(0))
                plsc.store_scatter(out_v, [idx], carry, mask=m)
                return carry

            carry = lax.fori_loop(0, 16, tail, carry)

            out_cps = [
                pltpu.async_copy(
                    out_v.at[pl.ds(rl * _SEG, _SEG)],
                    y_hbm.at[r0 + rl, pl.ds(c0, _SEG)],
                    sem,
                )
                for rl in range(16)
            ]
            for cp in out_cps:
                cp.wait()
            return carry

        lax.fori_loop(0, _NSEG, per_seg, jnp.zeros((16,), jnp.float32))
        return 0

    lax.fori_loop(0, _GROUPS, per_group, 0)


def kernel(x):
    mesh = plsc.VectorSubcoreMesh(core_axis_name="c", subcore_axis_name="s")
    k = functools.partial(
        pl.kernel,
        mesh=mesh,
        out_type=jax.ShapeDtypeStruct((_B, _N), jnp.float32),
        scratch_types=[
            pltpu.VMEM((16 * _SEG,), jnp.float32),
            pltpu.VMEM((16 * _SEG,), jnp.float32),
            pltpu.SemaphoreType.DMA,
        ],
        compiler_params=pltpu.CompilerParams(needs_layout_passes=False),
    )(_sc_body)
    return k(x)


# speedup vs baseline: 3.0514x; 1.5943x over previous
import functools

import jax
import jax.numpy as jnp
from jax import lax
from jax.experimental import pallas as pl
from jax.experimental.pallas import tpu as pltpu
from jax.experimental.pallas import tpu_sc as plsc

_B = 4096
_N = 8192
_NC = 2
_NS = 16
_NW = _NC * _NS
_GROUPS = _B // (_NW * 16)
_SEG = 1024
_NSEG = _N // _SEG
_TOT = _GROUPS * _NSEG
_UNROLL = 8


def _sc_body(x_hbm, y_hbm, in0, in1, out0, out1, si0, si1, so0, so1):
    wid = lax.axis_index("s") * _NC + lax.axis_index("c")
    lane = lax.broadcasted_iota(jnp.int32, (16,), 0)
    rowbase = lane * _SEG
    ins, outs = (in0, in1), (out0, out1)
    isems, osems = (si0, si1), (so0, so1)

    def coords(it):
        g = it // _NSEG
        s = lax.rem(it, _NSEG)
        r0 = (wid * _GROUPS + g) * 16
        c0 = (_NSEG - 1 - s) * _SEG
        return s, r0, c0

    def fire_in(it, q):
        s, r0, c0 = coords(it)
        for rl in range(16):
            pltpu.async_copy(
                x_hbm.at[r0 + rl, pl.ds(c0, _SEG)],
                ins[q].at[pl.ds(rl * _SEG, _SEG)],
                isems[q],
            )

    def body(it, q, carry):
        in_v, out_v = ins[q], outs[q]
        s, r0, c0 = coords(it)
        carry = jnp.where(s == 0, jnp.zeros((16,), jnp.float32), carry)

        @pl.when(it + 1 < _TOT)
        def _():
            fire_in(it + 1, 1 - q)

        for rl in range(16):
            pltpu.make_async_copy(
                x_hbm.at[r0 + rl, pl.ds(c0, _SEG)],
                in_v.at[pl.ds(rl * _SEG, _SEG)],
                isems[q],
            ).wait()

        @pl.when(it >= 2)
        def _():
            for rl in range(16):
                pltpu.make_async_copy(
                    x_hbm.at[r0 + rl, pl.ds(c0, _SEG)],
                    out_v.at[pl.ds(rl * _SEG, _SEG)],
                    osems[q],
                ).wait()

        def head(hi, carry):
            j = (_SEG + 14 - hi) - lane
            m = j <= _SEG - 1
            idx = rowbase + jnp.minimum(j, _SEG - 1)
            v = plsc.load_gather(in_v, [idx], mask=m)
            carry = carry + jnp.where(m, v, jnp.float32(0))
            plsc.store_scatter(out_v, [idx], carry, mask=m)
            return carry

        carry = lax.fori_loop(0, 15, head, carry)

        def main(_, state):
            carry, idx = state
            vs = [plsc.load_gather(in_v, [idx - k]) for k in range(_UNROLL)]
            pref = [vs[0]]
            for k in range(1, _UNROLL):
                pref.append(pref[-1] + vs[k])
            for k in range(_UNROLL):
                plsc.store_scatter(out_v, [idx - k], carry + pref[k])
            return carry + pref[-1], idx - _UNROLL

        carry, _ = lax.fori_loop(
            0, (_SEG - 16) // _UNROLL, main,
            (carry, rowbase + (_SEG - 1) - lane),
        )

        def tail(ti, carry):
            j = (15 - ti) - lane
            m = j >= 0
            idx = rowbase + jnp.maximum(j, 0)
            v = plsc.load_gather(in_v, [idx], mask=m)
            carry = carry + jnp.where(m, v, jnp.float32(0))
            plsc.store_scatter(out_v, [idx], carry, mask=m)
            return carry

        carry = lax.fori_loop(0, 16, tail, carry)

        for rl in range(16):
            pltpu.async_copy(
                out_v.at[pl.ds(rl * _SEG, _SEG)],
                y_hbm.at[r0 + rl, pl.ds(c0, _SEG)],
                osems[q],
            )
        return carry

    fire_in(0, 0)

    def pair(ip, carry):
        it = ip * 2
        carry = body(it, 0, carry)
        carry = body(it + 1, 1, carry)
        return carry

    lax.fori_loop(0, _TOT // 2, pair, jnp.zeros((16,), jnp.float32))

    for q, it in ((0, _TOT - 2), (1, _TOT - 1)):
        s, r0, c0 = coords(it)
        for rl in range(16):
            pltpu.make_async_copy(
                x_hbm.at[r0 + rl, pl.ds(c0, _SEG)],
                outs[q].at[pl.ds(rl * _SEG, _SEG)],
                osems[q],
            ).wait()


def kernel(x):
    mesh = plsc.VectorSubcoreMesh(core_axis_name="c", subcore_axis_name="s")
    k = functools.partial(
        pl.kernel,
        mesh=mesh,
        out_type=jax.ShapeDtypeStruct((_B, _N), jnp.float32),
        scratch_types=[
            pltpu.VMEM((16 * _SEG,), jnp.float32),
            pltpu.VMEM((16 * _SEG,), jnp.float32),
            pltpu.VMEM((16 * _SEG,), jnp.float32),
            pltpu.VMEM((16 * _SEG,), jnp.float32),
            pltpu.SemaphoreType.DMA,
            pltpu.SemaphoreType.DMA,
            pltpu.SemaphoreType.DMA,
            pltpu.SemaphoreType.DMA,
        ],
        compiler_params=pltpu.CompilerParams(needs_layout_passes=False),
    )(_sc_body)
    return k(x)


# speedup vs baseline: 3.4266x; 1.1230x over previous
import functools

import jax
import jax.numpy as jnp
from jax import lax
from jax.experimental import pallas as pl
from jax.experimental.pallas import tpu as pltpu
from jax.experimental.pallas import tpu_sc as plsc

_B = 4096
_N = 8192
_NC = 2
_NS = 16
_NW = _NC * _NS
_GROUPS = _B // (_NW * 16)
_SEG = 1024
_NSEG = _N // _SEG
_TOT = _GROUPS * _NSEG
_UNROLL = 16


def _sc_body(x_hbm, y_hbm, in0, in1, out0, out1, si0, si1, so0, so1):
    wid = lax.axis_index("s") * _NC + lax.axis_index("c")
    lane = lax.broadcasted_iota(jnp.int32, (16,), 0)
    rowbase = lane * _SEG
    ins, outs = (in0, in1), (out0, out1)
    isems, osems = (si0, si1), (so0, so1)

    def coords(it):
        g = it // _NSEG
        s = lax.rem(it, _NSEG)
        r0 = (wid * _GROUPS + g) * 16
        c0 = (_NSEG - 1 - s) * _SEG
        return s, r0, c0

    def fire_in(it, q):
        s, r0, c0 = coords(it)
        for rl in range(16):
            pltpu.async_copy(
                x_hbm.at[r0 + rl, pl.ds(c0, _SEG)],
                ins[q].at[pl.ds(rl * _SEG, _SEG)],
                isems[q],
            )

    def body(it, q, carry):
        in_v, out_v = ins[q], outs[q]
        s, r0, c0 = coords(it)
        carry = jnp.where(s == 0, jnp.zeros((16,), jnp.float32), carry)

        @pl.when(it + 1 < _TOT)
        def _():
            fire_in(it + 1, 1 - q)

        for rl in range(16):
            pltpu.make_async_copy(
                x_hbm.at[r0 + rl, pl.ds(c0, _SEG)],
                in_v.at[pl.ds(rl * _SEG, _SEG)],
                isems[q],
            ).wait()

        @pl.when(it >= 2)
        def _():
            for rl in range(16):
                pltpu.make_async_copy(
                    x_hbm.at[r0 + rl, pl.ds(c0, _SEG)],
                    out_v.at[pl.ds(rl * _SEG, _SEG)],
                    osems[q],
                ).wait()

        def head(hi, carry):
            j = (_SEG + 14 - hi) - lane
            m = j <= _SEG - 1
            idx = rowbase + jnp.minimum(j, _SEG - 1)
            v = plsc.load_gather(in_v, [idx], mask=m)
            carry = carry + jnp.where(m, v, jnp.float32(0))
            plsc.store_scatter(out_v, [idx], carry, mask=m)
            return carry

        carry = lax.fori_loop(0, 15, head, carry)

        def main(_, state):
            carry, idx = state
            vs = [plsc.load_gather(in_v, [idx - k]) for k in range(_UNROLL)]
            pref = [vs[0]]
            for k in range(1, _UNROLL):
                pref.append(pref[-1] + vs[k])
            for k in range(_UNROLL):
                plsc.store_scatter(out_v, [idx - k], carry + pref[k])
            return carry + pref[-1], idx - _UNROLL

        carry, _ = lax.fori_loop(
            0, (_SEG - 16) // _UNROLL, main,
            (carry, rowbase + (_SEG - 1) - lane),
        )

        def tail(ti, carry):
            j = (15 - ti) - lane
            m = j >= 0
            idx = rowbase + jnp.maximum(j, 0)
            v = plsc.load_gather(in_v, [idx], mask=m)
            carry = carry + jnp.where(m, v, jnp.float32(0))
            plsc.store_scatter(out_v, [idx], carry, mask=m)
            return carry

        carry = lax.fori_loop(0, 16, tail, carry)

        for rl in range(16):
            pltpu.async_copy(
                out_v.at[pl.ds(rl * _SEG, _SEG)],
                y_hbm.at[r0 + rl, pl.ds(c0, _SEG)],
                osems[q],
            )
        return carry

    fire_in(0, 0)

    def pair(ip, carry):
        it = ip * 2
        carry = body(it, 0, carry)
        carry = body(it + 1, 1, carry)
        return carry

    lax.fori_loop(0, _TOT // 2, pair, jnp.zeros((16,), jnp.float32))

    for q, it in ((0, _TOT - 2), (1, _TOT - 1)):
        s, r0, c0 = coords(it)
        for rl in range(16):
            pltpu.make_async_copy(
                x_hbm.at[r0 + rl, pl.ds(c0, _SEG)],
                outs[q].at[pl.ds(rl * _SEG, _SEG)],
                osems[q],
            ).wait()


def kernel(x):
    mesh = plsc.VectorSubcoreMesh(core_axis_name="c", subcore_axis_name="s")
    k = functools.partial(
        pl.kernel,
        mesh=mesh,
        out_type=jax.ShapeDtypeStruct((_B, _N), jnp.float32),
        scratch_types=[
            pltpu.VMEM((16 * _SEG,), jnp.float32),
            pltpu.VMEM((16 * _SEG,), jnp.float32),
            pltpu.VMEM((16 * _SEG,), jnp.float32),
            pltpu.VMEM((16 * _SEG,), jnp.float32),
            pltpu.SemaphoreType.DMA,
            pltpu.SemaphoreType.DMA,
            pltpu.SemaphoreType.DMA,
            pltpu.SemaphoreType.DMA,
        ],
        compiler_params=pltpu.CompilerParams(needs_layout_passes=False),
    )(_sc_body)
    return k(x)
